# baseline (speedup 1.0000x reference)
import functools

import jax
import jax.numpy as jnp
from jax import lax
from jax.experimental import pallas as pl
from jax.experimental.pallas import tpu as pltpu
from jax.experimental.pallas import tpu_sc as plsc

NC = 2
NS = 16
NW = NC * NS


def _dinv_from(g_ref):
    deg = g_ref[:, 0] + g_ref[:, 1]
    return jnp.where(deg > 0, lax.rsqrt(deg), 0.0)


def _dot(a, b):
    return jnp.dot(a, b, preferred_element_type=jnp.float32)


@functools.lru_cache(maxsize=None)
def _build(N, D, E):
    EPW = E // NW
    K = 40
    assert EPW % K == 0
    NCH = EPW // K
    NBUF = 5
    NIB = 10
    assert NCH % NIB == 0
    NG2 = NCH // NIB
    NG = NCH // NBUF
    KD = 80
    assert EPW % KD == 0
    NCHD = EPW // KD
    assert NCHD % NBUF == 0
    NGD = NCHD // NBUF
    ZC = 1000
    NZT = N // ZC
    ZR = K
    NCHZ = N // ZR
    JZ = -(-NCHZ // NS)
    assert N % ZR == 0 and ZR % 8 == 0

    mesh = plsc.VectorSubcoreMesh(core_axis_name="c", subcore_axis_name="s")

    @functools.partial(
        pl.kernel,
        out_type=jax.ShapeDtypeStruct((2 * N,), jnp.float32),
        mesh=mesh,
        scratch_types=[
            pltpu.VMEM((NBUF, KD), jnp.int32),
            pltpu.VMEM((KD,), jnp.float32),
            pltpu.VMEM((ZC,), jnp.float32),
            pltpu.VMEM_SHARED((N,), jnp.float32),
            pltpu.SemaphoreType.DMA((NBUF,)),
            pltpu.SemaphoreType.DMA((NBUF,)),
        ],
    )
    def deg_kernel(edges_hbm, zeros_hbm, ones_hbm, out_hbm, cidx_v, ones_v,
                   buf_v, deg_sh, isem, ssem):
        c = lax.axis_index("c")
        s = lax.axis_index("s")
        pltpu.sync_copy(ones_hbm, ones_v)

        @pl.when(s < NZT)
        def _():
            pltpu.sync_copy(zeros_hbm, buf_v)
            pltpu.sync_copy(buf_v, deg_sh.at[pl.ds(s * ZC, ZC)])

        plsc.subcore_barrier()
        base = (s * NC + c) * EPW

        def didx(j, b):
            return pltpu.make_async_copy(
                edges_hbm.at[pl.ds(E + base + j * KD, KD)], cidx_v.at[b],
                isem.at[b])

        def dscat(b):
            return pltpu.make_async_copy(
                ones_v, deg_sh.at[cidx_v.at[b]], ssem.at[b])

        for b in range(2):
            didx(b, b).start()

        def body(g, carry):
            for b in range(NBUF):
                j = g * NBUF + b
                ba = (b + 2) % NBUF

                def stage_a():
                    dscat(ba).wait()
                    didx(j + 2, ba).start()

                if b < 3:
                    @pl.when(g >= 1)
                    def _():
                        stage_a()

                    @pl.when(g == 0)
                    def _():
                        didx(j + 2, ba).start()
                else:
                    @pl.when(g < NGD - 1)
                    def _():
                        stage_a()

                didx(j, b).wait()
                dscat(b).start(add=True)
            return carry

        lax.fori_loop(0, NGD, body, 0)
        for b in range(NBUF):
            dscat(b).wait()
        plsc.subcore_barrier()

        @pl.when(s < NZT)
        def _():
            pltpu.sync_copy(deg_sh.at[pl.ds(s * ZC, ZC)], buf_v)
            pltpu.sync_copy(buf_v, out_hbm.at[pl.ds(c * N + s * ZC, ZC)])

    @functools.partial(
        pl.kernel,
        out_type=jax.ShapeDtypeStruct((2, N, D), jnp.float32),
        mesh=mesh,
        compiler_params=pltpu.CompilerParams(use_tc_tiling_on_sc=True),
        scratch_types=[
            pltpu.VMEM((NIB, K), jnp.int32),
            pltpu.VMEM((NIB, K), jnp.int32),
            pltpu.VMEM((NBUF, K, D), jnp.float32),
            pltpu.VMEM_SHARED((N, D), jnp.float32),
            pltpu.SemaphoreType.DMA((NIB,)),
            pltpu.SemaphoreType.DMA((NBUF,)),
            pltpu.SemaphoreType.DMA((NBUF,)),
        ],
    )
    def prop_kernel(h_hbm, edges_hbm, zeros_hbm, out_hbm,
                    ridx_v, cidx_v, rows_v, acc_sh, isem, gsem, ssem):
        c = lax.axis_index("c")
        s = lax.axis_index("s")
        wid = s * NC + c
        base = wid * EPW
        pltpu.sync_copy(zeros_hbm, rows_v.at[0])

        def zbody(k, carry):
            ch = s + k * NS

            @pl.when(ch < NCHZ)
            def _():
                pltpu.sync_copy(rows_v.at[0], acc_sh.at[pl.ds(ch * ZR, ZR)])

            return carry

        lax.fori_loop(0, JZ, zbody, 0)
        plsc.subcore_barrier()

        def idx(j, bi):
            e0 = base + j * K
            return (pltpu.make_async_copy(edges_hbm.at[pl.ds(e0, K)],
                                          ridx_v.at[bi], isem.at[bi]),
                    pltpu.make_async_copy(edges_hbm.at[pl.ds(E + e0, K)],
                                          cidx_v.at[bi], isem.at[bi]))

        def gath(bi, b):
            return pltpu.make_async_copy(
                h_hbm.at[ridx_v.at[bi]], rows_v.at[b], gsem.at[b])

        def scat(bi, b):
            return pltpu.make_async_copy(
                rows_v.at[b], acc_sh.at[cidx_v.at[bi]], ssem.at[b])

        def idx_start(j, bi):
            d1, d2 = idx(j, bi)
            d1.start()
            d2.start()

        def idx_wait(j, bi):
            d1, d2 = idx(j, bi)
            d1.wait()
            d2.wait()

        for t in range(5):
            idx_start(t, t)
        for t in range(2):
            idx_wait(t, t)
            gath(t, t % NBUF).start()

        def body(g, carry):
            for b in range(NIB):
                j = g * NIB + b
                if b < 5:
                    idx_start(j + 5, (b + 5) % NIB)
                else:
                    @pl.when(g < NG2 - 1)
                    def _():
                        idx_start(j + 5, (b + 5) % NIB)

                b2 = (b + 2) % NBUF
                bi2 = (b + 2) % NIB

                def stage_g():
                    scat(bi2, b2).wait()
                    idx_wait(j + 2, bi2)
                    gath(bi2, b2).start()

                def stage_g0():
                    idx_wait(j + 2, bi2)
                    gath(bi2, b2).start()

                if b < 3:
                    @pl.when(g >= 1)
                    def _():
                        stage_g()

                    @pl.when(g == 0)
                    def _():
                        stage_g0()
                elif b < 8:
                    stage_g()
                else:
                    @pl.when(g < NG2 - 1)
                    def _():
                        stage_g()

                gath(b % NIB, b % NBUF).wait()
                scat(b % NIB, b % NBUF).start(add=True)
            return carry

        lax.fori_loop(0, NG2, body, 0)
        for t in range(NCH - 5, NCH):
            scat(t % NIB, t % NBUF).wait()
        plsc.subcore_barrier()

        def obody(k, carry):
            ch = s + k * NS

            @pl.when(ch < NCHZ)
            def _():
                pltpu.sync_copy(acc_sh.at[pl.ds(ch * ZR, ZR)], rows_v.at[0])
                pltpu.sync_copy(rows_v.at[0], out_hbm.at[c, pl.ds(ch * ZR, ZR)])

            return carry

        lax.fori_loop(0, JZ, obody, 0)

    RB = 2000
    assert N % RB == 0
    grid = (N // RB,)

    def mm_scale_body(x_ref, w_ref, g_ref, o_ref):
        dinv = _dinv_from(g_ref)
        o_ref[...] = _dot(x_ref[...], w_ref[...]) * dinv[:, None]

    mm_scale = pl.pallas_call(
        mm_scale_body,
        grid=grid,
        in_specs=[
            pl.BlockSpec((RB, D), lambda i: (i, 0)),
            pl.BlockSpec((D, D), lambda i: (0, 0)),
            pl.BlockSpec((RB, 2), lambda i: (i, 0)),
        ],
        out_specs=pl.BlockSpec((RB, D), lambda i: (i, 0)),
        out_shape=jax.ShapeDtypeStruct((N, D), jnp.float32),
    )

    def mid_body(pp_ref, g_ref, b_ref, w_ref, o_ref):
        dinv = _dinv_from(g_ref)
        h1 = (pp_ref[0] + pp_ref[1]) * dinv[:, None] + b_ref[...]
        o_ref[...] = _dot(h1, w_ref[...]) * dinv[:, None]

    mid = pl.pallas_call(
        mid_body,
        grid=grid,
        in_specs=[
            pl.BlockSpec((2, RB, D), lambda i: (0, i, 0)),
            pl.BlockSpec((RB, 2), lambda i: (i, 0)),
            pl.BlockSpec((1, D), lambda i: (0, 0)),
            pl.BlockSpec((D, D), lambda i: (0, 0)),
        ],
        out_specs=pl.BlockSpec((RB, D), lambda i: (i, 0)),
        out_shape=jax.ShapeDtypeStruct((N, D), jnp.float32),
    )

    def fin_body(pp_ref, g_ref, b_ref, o_ref):
        dinv = _dinv_from(g_ref)
        o_ref[...] = (pp_ref[0] + pp_ref[1]) * dinv[:, None] + b_ref[...]

    fin = pl.pallas_call(
        fin_body,
        grid=grid,
        in_specs=[
            pl.BlockSpec((2, RB, D), lambda i: (0, i, 0)),
            pl.BlockSpec((RB, 2), lambda i: (i, 0)),
            pl.BlockSpec((1, D), lambda i: (0, 0)),
        ],
        out_specs=pl.BlockSpec((RB, D), lambda i: (i, 0)),
        out_shape=jax.ShapeDtypeStruct((N, D), jnp.float32),
    )

    return deg_kernel, prop_kernel, mm_scale, mid, fin


def kernel(x, edge_index, W1, b1, W2, b2):
    N, D = x.shape
    E = edge_index.shape[1]
    deg_kernel, prop_kernel, mm_scale, mid, fin = _build(N, D, E)

    EPW = E // NW
    K = 40
    NCH = EPW // K
    edges = edge_index.reshape(2 * E)
    zeros_nd = jnp.zeros((K, D), jnp.float32)
    zeros_n = jnp.zeros((1000,), jnp.float32)
    ones_k = jnp.ones((80,), jnp.float32)

    degp = deg_kernel(edges, zeros_n, ones_k)
    degt = degp.reshape(2, N).T

    h1 = mm_scale(x, W1, degt)
    pp1 = prop_kernel(h1, edges, zeros_nd)
    h2 = mid(pp1, degt, b1.reshape(1, D), W2)
    pp2 = prop_kernel(h2, edges, zeros_nd)
    return fin(pp2, degt, b2.reshape(1, D))


# speedup vs baseline: 1.0147x; 1.0147x over previous
import functools

import jax
import jax.numpy as jnp
from jax import lax
from jax.experimental import pallas as pl
from jax.experimental.pallas import tpu as pltpu
from jax.experimental.pallas import tpu_sc as plsc

NC = 2
NS = 16
NW = NC * NS


def _dinv_from(g_ref):
    deg = g_ref[:, 0] + g_ref[:, 1]
    return jnp.where(deg > 0, lax.rsqrt(deg), 0.0)


def _dot(a, b):
    return jnp.dot(a, b, preferred_element_type=jnp.float32)


@functools.lru_cache(maxsize=None)
def _build(N, D, E):
    EPW = E // NW
    K = 40
    assert EPW % K == 0
    NCH = EPW // K
    NBUF = 5
    NIB = 10
    assert NCH % NIB == 0
    NG2 = NCH // NIB
    NG = NCH // NBUF
    KD = 80
    assert EPW % KD == 0
    NCHD = EPW // KD
    assert NCHD % NBUF == 0
    NGD = NCHD // NBUF
    ZC = 1000
    NZT = N // ZC
    ZR = K
    NCHZ = N // ZR
    JZ = -(-NCHZ // NS)
    assert N % ZR == 0 and ZR % 8 == 0

    mesh = plsc.VectorSubcoreMesh(core_axis_name="c", subcore_axis_name="s")

    @functools.partial(
        pl.kernel,
        out_type=jax.ShapeDtypeStruct((2 * N,), jnp.float32),
        mesh=mesh,
        scratch_types=[
            pltpu.VMEM((NBUF, KD), jnp.int32),
            pltpu.VMEM((KD,), jnp.float32),
            pltpu.VMEM((ZC,), jnp.float32),
            pltpu.VMEM_SHARED((N,), jnp.float32),
            pltpu.SemaphoreType.DMA((NBUF,)),
            pltpu.SemaphoreType.DMA((NBUF,)),
        ],
    )
    def deg_kernel(edges_hbm, zeros_hbm, ones_hbm, out_hbm, cidx_v, ones_v,
                   buf_v, deg_sh, isem, ssem):
        c = lax.axis_index("c")
        s = lax.axis_index("s")
        pltpu.sync_copy(ones_hbm, ones_v)

        @pl.when(s < NZT)
        def _():
            pltpu.sync_copy(zeros_hbm, buf_v)
            pltpu.sync_copy(buf_v, deg_sh.at[pl.ds(s * ZC, ZC)])

        plsc.subcore_barrier()
        base = (s * NC + c) * EPW

        def didx(j, b):
            return pltpu.make_async_copy(
                edges_hbm.at[pl.ds(E + base + j * KD, KD)], cidx_v.at[b],
                isem.at[b])

        def dscat(b):
            return pltpu.make_async_copy(
                ones_v, deg_sh.at[cidx_v.at[b]], ssem.at[b])

        for b in range(3):
            didx(b, b).start()

        def body(g, carry):
            for b in range(NBUF):
                j = g * NBUF + b
                ba = (b + 3) % NBUF

                if b < 2:
                    @pl.when(g >= 1)
                    def _():
                        dscat(ba).wait()

                    didx(j + 3, ba).start()
                else:
                    dscat(ba).wait()

                    @pl.when(g < NGD - 1)
                    def _():
                        didx(j + 3, ba).start()

                didx(j, b).wait()
                dscat(b).start(add=True)
            return carry

        lax.fori_loop(0, NGD, body, 0)
        for t in range(NCHD - 2, NCHD):
            dscat(t % NBUF).wait()
        plsc.subcore_barrier()

        @pl.when(s < NZT)
        def _():
            pltpu.sync_copy(deg_sh.at[pl.ds(s * ZC, ZC)], buf_v)
            pltpu.sync_copy(buf_v, out_hbm.at[pl.ds(c * N + s * ZC, ZC)])

    @functools.partial(
        pl.kernel,
        out_type=jax.ShapeDtypeStruct((2, N, D), jnp.float32),
        mesh=mesh,
        scratch_types=[
            pltpu.VMEM((NIB, K), jnp.int32),
            pltpu.VMEM((NIB, K), jnp.int32),
            pltpu.VMEM((NBUF, K, D), jnp.float32),
            pltpu.VMEM_SHARED((N, D), jnp.float32),
            pltpu.SemaphoreType.DMA((NIB,)),
            pltpu.SemaphoreType.DMA((NBUF,)),
            pltpu.SemaphoreType.DMA((NBUF,)),
        ],
    )
    def prop_kernel(h_hbm, edges_hbm, zeros_hbm, out_hbm,
                    ridx_v, cidx_v, rows_v, acc_sh, isem, gsem, ssem):
        c = lax.axis_index("c")
        s = lax.axis_index("s")
        wid = s * NC + c
        base = wid * EPW
        pltpu.sync_copy(zeros_hbm, rows_v.at[0])

        def zbody(k, carry):
            ch = s + k * NS

            @pl.when(ch < NCHZ)
            def _():
                pltpu.sync_copy(rows_v.at[0], acc_sh.at[pl.ds(ch * ZR, ZR)])

            return carry

        lax.fori_loop(0, JZ, zbody, 0)
        plsc.subcore_barrier()

        def idx(j, bi):
            e0 = base + j * K
            return (pltpu.make_async_copy(edges_hbm.at[pl.ds(e0, K)],
                                          ridx_v.at[bi], isem.at[bi]),
                    pltpu.make_async_copy(edges_hbm.at[pl.ds(E + e0, K)],
                                          cidx_v.at[bi], isem.at[bi]))

        def gath(bi, b):
            return pltpu.make_async_copy(
                h_hbm.at[ridx_v.at[bi]], rows_v.at[b], gsem.at[b])

        def scat(bi, b):
            return pltpu.make_async_copy(
                rows_v.at[b], acc_sh.at[cidx_v.at[bi]], ssem.at[b])

        def idx_start(j, bi):
            d1, d2 = idx(j, bi)
            d1.start()
            d2.start()

        def idx_wait(j, bi):
            d1, d2 = idx(j, bi)
            d1.wait()
            d2.wait()

        for t in range(5):
            idx_start(t, t)
        for t in range(2):
            idx_wait(t, t)
            gath(t, t % NBUF).start()

        def body(g, carry):
            for b in range(NIB):
                j = g * NIB + b
                if b < 5:
                    idx_start(j + 5, (b + 5) % NIB)
                else:
                    @pl.when(g < NG2 - 1)
                    def _():
                        idx_start(j + 5, (b + 5) % NIB)

                b2 = (b + 2) % NBUF
                bi2 = (b + 2) % NIB

                def stage_g():
                    scat(bi2, b2).wait()
                    idx_wait(j + 2, bi2)
                    gath(bi2, b2).start()

                def stage_g0():
                    idx_wait(j + 2, bi2)
                    gath(bi2, b2).start()

                if b < 3:
                    @pl.when(g >= 1)
                    def _():
                        stage_g()

                    @pl.when(g == 0)
                    def _():
                        stage_g0()
                elif b < 8:
                    stage_g()
                else:
                    @pl.when(g < NG2 - 1)
                    def _():
                        stage_g()

                gath(b % NIB, b % NBUF).wait()
                scat(b % NIB, b % NBUF).start(add=True)
            return carry

        lax.fori_loop(0, NG2, body, 0)
        for t in range(NCH - 5, NCH):
            scat(t % NIB, t % NBUF).wait()
        plsc.subcore_barrier()

        def obody(k, carry):
            ch = s + k * NS

            @pl.when(ch < NCHZ)
            def _():
                pltpu.sync_copy(acc_sh.at[pl.ds(ch * ZR, ZR)], rows_v.at[0])
                pltpu.sync_copy(rows_v.at[0], out_hbm.at[c, pl.ds(ch * ZR, ZR)])

            return carry

        lax.fori_loop(0, JZ, obody, 0)

    RB = 2000
    assert N % RB == 0
    grid = (N // RB,)

    def mm_scale_body(x_ref, w_ref, g_ref, o_ref):
        dinv = _dinv_from(g_ref)
        o_ref[...] = _dot(x_ref[...], w_ref[...]) * dinv[:, None]

    mm_scale = pl.pallas_call(
        mm_scale_body,
        grid=grid,
        in_specs=[
            pl.BlockSpec((RB, D), lambda i: (i, 0)),
            pl.BlockSpec((D, D), lambda i: (0, 0)),
            pl.BlockSpec((RB, 2), lambda i: (i, 0)),
        ],
        out_specs=pl.BlockSpec((RB, D), lambda i: (i, 0)),
        out_shape=jax.ShapeDtypeStruct((N, D), jnp.float32),
    )

    def mid_body(pp_ref, g_ref, b_ref, w_ref, o_ref):
        dinv = _dinv_from(g_ref)
        h1 = (pp_ref[0] + pp_ref[1]) * dinv[:, None] + b_ref[...]
        o_ref[...] = _dot(h1, w_ref[...]) * dinv[:, None]

    mid = pl.pallas_call(
        mid_body,
        grid=grid,
        in_specs=[
            pl.BlockSpec((2, RB, D), lambda i: (0, i, 0)),
            pl.BlockSpec((RB, 2), lambda i: (i, 0)),
            pl.BlockSpec((1, D), lambda i: (0, 0)),
            pl.BlockSpec((D, D), lambda i: (0, 0)),
        ],
        out_specs=pl.BlockSpec((RB, D), lambda i: (i, 0)),
        out_shape=jax.ShapeDtypeStruct((N, D), jnp.float32),
    )

    def fin_body(pp_ref, g_ref, b_ref, o_ref):
        dinv = _dinv_from(g_ref)
        o_ref[...] = (pp_ref[0] + pp_ref[1]) * dinv[:, None] + b_ref[...]

    fin = pl.pallas_call(
        fin_body,
        grid=grid,
        in_specs=[
            pl.BlockSpec((2, RB, D), lambda i: (0, i, 0)),
            pl.BlockSpec((RB, 2), lambda i: (i, 0)),
            pl.BlockSpec((1, D), lambda i: (0, 0)),
        ],
        out_specs=pl.BlockSpec((RB, D), lambda i: (i, 0)),
        out_shape=jax.ShapeDtypeStruct((N, D), jnp.float32),
    )

    return deg_kernel, prop_kernel, mm_scale, mid, fin


def kernel(x, edge_index, W1, b1, W2, b2):
    N, D = x.shape
    E = edge_index.shape[1]
    deg_kernel, prop_kernel, mm_scale, mid, fin = _build(N, D, E)

    EPW = E // NW
    K = 40
    NCH = EPW // K
    edges = edge_index.reshape(2 * E)
    zeros_nd = jnp.zeros((K, D), jnp.float32)
    zeros_n = jnp.zeros((1000,), jnp.float32)
    ones_k = jnp.ones((80,), jnp.float32)

    degp = deg_kernel(edges, zeros_n, ones_k)
    degt = degp.reshape(2, N).T

    h1 = mm_scale(x, W1, degt)
    pp1 = prop_kernel(h1, edges, zeros_nd)
    h2 = mid(pp1, degt, b1.reshape(1, D), W2)
    pp2 = prop_kernel(h2, edges, zeros_nd)
    return fin(pp2, degt, b2.reshape(1, D))


# speedup vs baseline: 1.0943x; 1.0784x over previous
import functools

import jax
import jax.numpy as jnp
from jax import lax
from jax.experimental import pallas as pl
from jax.experimental.pallas import tpu as pltpu
from jax.experimental.pallas import tpu_sc as plsc

NC = 2
NS = 16
NW = NC * NS


def _dinv_from(g_ref):
    deg = g_ref[:, 0] + g_ref[:, 1]
    return jnp.where(deg > 0, lax.rsqrt(deg), 0.0)


def _dot(a, b):
    return jnp.dot(a, b, preferred_element_type=jnp.float32)


@functools.lru_cache(maxsize=None)
def _build(N, D, E):
    EPW = E // NW
    K = 40
    assert EPW % K == 0
    NCH = EPW // K
    NBUF = 5
    NIB = 10
    assert NCH % NIB == 0
    NG2 = NCH // NIB
    NG = NCH // NBUF
    KD = 80
    assert EPW % KD == 0
    NCHD = EPW // KD
    assert NCHD % NBUF == 0
    NGD = NCHD // NBUF
    ZC = 1000
    NZT = N // ZC
    ZR = K
    NCHZ = N // ZR
    JZ = -(-NCHZ // NS)
    assert N % ZR == 0 and ZR % 8 == 0

    mesh = plsc.VectorSubcoreMesh(core_axis_name="c", subcore_axis_name="s")

    @functools.partial(
        pl.kernel,
        out_type=jax.ShapeDtypeStruct((2 * N,), jnp.float32),
        mesh=mesh,
        scratch_types=[
            pltpu.VMEM((NBUF, KD), jnp.int32),
            pltpu.VMEM((KD,), jnp.float32),
            pltpu.VMEM((ZC,), jnp.float32),
            pltpu.VMEM_SHARED((N,), jnp.float32),
            pltpu.SemaphoreType.DMA((NBUF,)),
            pltpu.SemaphoreType.DMA((NBUF,)),
        ],
    )
    def deg_kernel(edges_hbm, zeros_hbm, ones_hbm, out_hbm, cidx_v, ones_v,
                   buf_v, deg_sh, isem, ssem):
        c = lax.axis_index("c")
        s = lax.axis_index("s")
        pltpu.sync_copy(ones_hbm, ones_v)

        @pl.when(s < NZT)
        def _():
            pltpu.sync_copy(zeros_hbm, buf_v)
            pltpu.sync_copy(buf_v, deg_sh.at[pl.ds(s * ZC, ZC)])

        plsc.subcore_barrier()
        base = (s * NC + c) * EPW

        def didx(j, b):
            return pltpu.make_async_copy(
                edges_hbm.at[pl.ds(E + base + j * KD, KD)], cidx_v.at[b],
                isem.at[b])

        def dscat(b):
            return pltpu.make_async_copy(
                ones_v, deg_sh.at[cidx_v.at[b]], ssem.at[b])

        for b in range(3):
            didx(b, b).start()

        def body(g, carry):
            for b in range(NBUF):
                j = g * NBUF + b
                ba = (b + 3) % NBUF

                if b < 2:
                    @pl.when(g >= 1)
                    def _():
                        dscat(ba).wait()

                    didx(j + 3, ba).start()
                else:
                    dscat(ba).wait()

                    @pl.when(g < NGD - 1)
                    def _():
                        didx(j + 3, ba).start()

                didx(j, b).wait()
                dscat(b).start(add=True)
            return carry

        lax.fori_loop(0, NGD, body, 0)
        for t in range(NCHD - 2, NCHD):
            dscat(t % NBUF).wait()
        plsc.subcore_barrier()

        @pl.when(s < NZT)
        def _():
            pltpu.sync_copy(deg_sh.at[pl.ds(s * ZC, ZC)], buf_v)
            pltpu.sync_copy(buf_v, out_hbm.at[pl.ds(c * N + s * ZC, ZC)])

    @functools.partial(
        pl.kernel,
        out_type=jax.ShapeDtypeStruct((2, N, D), jnp.float32),
        mesh=mesh,
        scratch_types=[
            pltpu.VMEM((NIB, K), jnp.int32),
            pltpu.VMEM((NIB, K), jnp.int32),
            pltpu.VMEM((NBUF, K, D), jnp.float32),
            pltpu.VMEM_SHARED((N, D), jnp.float32),
            pltpu.SemaphoreType.DMA((NIB,)),
            pltpu.SemaphoreType.DMA((NBUF,)),
            pltpu.SemaphoreType.DMA((NBUF,)),
        ],
    )
    def prop_kernel(h_hbm, edges_hbm, zeros_hbm, out_hbm,
                    ridx_v, cidx_v, rows_v, acc_sh, isem, gsem, ssem):
        c = lax.axis_index("c")
        s = lax.axis_index("s")
        wid = s * NC + c
        base = wid * EPW
        pltpu.sync_copy(zeros_hbm, rows_v.at[0])

        def zbody(k, carry):
            ch = s + k * NS

            @pl.when(ch < NCHZ)
            def _():
                pltpu.sync_copy(rows_v.at[0], acc_sh.at[pl.ds(ch * ZR, ZR)])

            return carry

        lax.fori_loop(0, JZ, zbody, 0)
        plsc.subcore_barrier()

        def idx(j, bi):
            e0 = base + j * K
            return (pltpu.make_async_copy(edges_hbm.at[pl.ds(e0, K)],
                                          ridx_v.at[bi], isem.at[bi]),
                    pltpu.make_async_copy(edges_hbm.at[pl.ds(E + e0, K)],
                                          cidx_v.at[bi], isem.at[bi]))

        def gath(bi, b):
            return pltpu.make_async_copy(
                h_hbm.at[ridx_v.at[bi]], rows_v.at[b], gsem.at[b])

        def scat(bi, b):
            return pltpu.make_async_copy(
                rows_v.at[b], acc_sh.at[cidx_v.at[bi]], ssem.at[b])

        def idx_start(j, bi):
            d1, d2 = idx(j, bi)
            d1.start()
            d2.start()

        def idx_wait(j, bi):
            d1, d2 = idx(j, bi)
            d1.wait()
            d2.wait()

        for t in range(5):
            idx_start(t, t)
        for t in range(3):
            idx_wait(t, t)
            gath(t, t % NBUF).start()

        def body(g, carry):
            for b in range(NIB):
                j = g * NIB + b
                if b < 5:
                    idx_start(j + 5, (b + 5) % NIB)
                else:
                    @pl.when(g < NG2 - 1)
                    def _():
                        idx_start(j + 5, (b + 5) % NIB)

                b3 = (b + 3) % NBUF
                bi_d = (b + 8) % NIB
                bi_g = (b + 3) % NIB

                def gath_start():
                    idx_wait(j + 3, bi_g)
                    gath(bi_g, b3).start()

                if b < 2:
                    @pl.when(g >= 1)
                    def _():
                        scat(bi_d, b3).wait()

                    gath_start()
                elif b < 7:
                    scat(bi_d, b3).wait()
                    gath_start()
                else:
                    scat(bi_d, b3).wait()

                    @pl.when(g < NG2 - 1)
                    def _():
                        gath_start()

                gath(b % NIB, b % NBUF).wait()
                scat(b % NIB, b % NBUF).start(add=True)
            return carry

        lax.fori_loop(0, NG2, body, 0)
        for t in range(NCH - 2, NCH):
            scat(t % NIB, t % NBUF).wait()
        plsc.subcore_barrier()

        def obody(k, carry):
            ch = s + k * NS

            @pl.when(ch < NCHZ)
            def _():
                pltpu.sync_copy(acc_sh.at[pl.ds(ch * ZR, ZR)], rows_v.at[0])
                pltpu.sync_copy(rows_v.at[0], out_hbm.at[c, pl.ds(ch * ZR, ZR)])

            return carry

        lax.fori_loop(0, JZ, obody, 0)

    RB = 2000
    assert N % RB == 0
    grid = (N // RB,)

    def mm_scale_body(x_ref, w_ref, g_ref, o_ref):
        dinv = _dinv_from(g_ref)
        o_ref[...] = _dot(x_ref[...], w_ref[...]) * dinv[:, None]

    mm_scale = pl.pallas_call(
        mm_scale_body,
        grid=grid,
        in_specs=[
            pl.BlockSpec((RB, D), lambda i: (i, 0)),
            pl.BlockSpec((D, D), lambda i: (0, 0)),
            pl.BlockSpec((RB, 2), lambda i: (i, 0)),
        ],
        out_specs=pl.BlockSpec((RB, D), lambda i: (i, 0)),
        out_shape=jax.ShapeDtypeStruct((N, D), jnp.float32),
    )

    def mid_body(pp_ref, g_ref, b_ref, w_ref, o_ref):
        dinv = _dinv_from(g_ref)
        h1 = (pp_ref[0] + pp_ref[1]) * dinv[:, None] + b_ref[...]
        o_ref[...] = _dot(h1, w_ref[...]) * dinv[:, None]

    mid = pl.pallas_call(
        mid_body,
        grid=grid,
        in_specs=[
            pl.BlockSpec((2, RB, D), lambda i: (0, i, 0)),
            pl.BlockSpec((RB, 2), lambda i: (i, 0)),
            pl.BlockSpec((1, D), lambda i: (0, 0)),
            pl.BlockSpec((D, D), lambda i: (0, 0)),
        ],
        out_specs=pl.BlockSpec((RB, D), lambda i: (i, 0)),
        out_shape=jax.ShapeDtypeStruct((N, D), jnp.float32),
    )

    def fin_body(pp_ref, g_ref, b_ref, o_ref):
        dinv = _dinv_from(g_ref)
        o_ref[...] = (pp_ref[0] + pp_ref[1]) * dinv[:, None] + b_ref[...]

    fin = pl.pallas_call(
        fin_body,
        grid=grid,
        in_specs=[
            pl.BlockSpec((2, RB, D), lambda i: (0, i, 0)),
            pl.BlockSpec((RB, 2), lambda i: (i, 0)),
            pl.BlockSpec((1, D), lambda i: (0, 0)),
        ],
        out_specs=pl.BlockSpec((RB, D), lambda i: (i, 0)),
        out_shape=jax.ShapeDtypeStruct((N, D), jnp.float32),
    )

    return deg_kernel, prop_kernel, mm_scale, mid, fin


def kernel(x, edge_index, W1, b1, W2, b2):
    N, D = x.shape
    E = edge_index.shape[1]
    deg_kernel, prop_kernel, mm_scale, mid, fin = _build(N, D, E)

    EPW = E // NW
    K = 40
    NCH = EPW // K
    edges = edge_index.reshape(2 * E)
    zeros_nd = jnp.zeros((K, D), jnp.float32)
    zeros_n = jnp.zeros((1000,), jnp.float32)
    ones_k = jnp.ones((80,), jnp.float32)

    degp = deg_kernel(edges, zeros_n, ones_k)
    degt = degp.reshape(2, N).T

    h1 = mm_scale(x, W1, degt)
    pp1 = prop_kernel(h1, edges, zeros_nd)
    h2 = mid(pp1, degt, b1.reshape(1, D), W2)
    pp2 = prop_kernel(h2, edges, zeros_nd)
    return fin(pp2, degt, b2.reshape(1, D))


# speedup vs baseline: 1.1299x; 1.0326x over previous
import functools

import jax
import jax.numpy as jnp
from jax import lax
from jax.experimental import pallas as pl
from jax.experimental.pallas import tpu as pltpu
from jax.experimental.pallas import tpu_sc as plsc

NC = 2
NS = 16
NW = NC * NS


def _dinv_from(g_ref):
    deg = g_ref[:, 0] + g_ref[:, 1]
    return jnp.where(deg > 0, lax.rsqrt(deg), 0.0)


def _dot(a, b):
    return jnp.dot(a, b, preferred_element_type=jnp.float32)


@functools.lru_cache(maxsize=None)
def _build(N, D, E):
    EPW = E // NW
    K = 40
    assert EPW % K == 0
    NCH = EPW // K
    NBUF = 5
    NIB = 10
    assert NCH % NIB == 0
    NG2 = NCH // NIB
    NG = NCH // NBUF
    KD = 80
    assert EPW % KD == 0
    NCHD = EPW // KD
    assert NCHD % NBUF == 0
    NGD = NCHD // NBUF
    ZC = 1000
    NZT = N // ZC
    ZR = K
    NCHZ = N // ZR
    JZ = -(-NCHZ // NS)
    assert N % ZR == 0 and ZR % 8 == 0

    mesh = plsc.VectorSubcoreMesh(core_axis_name="c", subcore_axis_name="s")

    @functools.partial(
        pl.kernel,
        out_type=jax.ShapeDtypeStruct((2 * N,), jnp.float32),
        mesh=mesh,
        scratch_types=[
            pltpu.VMEM((NBUF, KD), jnp.int32),
            pltpu.VMEM((KD,), jnp.float32),
            pltpu.VMEM((ZC,), jnp.float32),
            pltpu.VMEM_SHARED((N,), jnp.float32),
            pltpu.SemaphoreType.DMA((NBUF,)),
            pltpu.SemaphoreType.DMA((NBUF,)),
        ],
    )
    def deg_kernel(edges_hbm, zeros_hbm, ones_hbm, out_hbm, cidx_v, ones_v,
                   buf_v, deg_sh, isem, ssem):
        c = lax.axis_index("c")
        s = lax.axis_index("s")
        pltpu.sync_copy(ones_hbm, ones_v)

        @pl.when(s < NZT)
        def _():
            pltpu.sync_copy(zeros_hbm, buf_v)
            pltpu.sync_copy(buf_v, deg_sh.at[pl.ds(s * ZC, ZC)])

        plsc.subcore_barrier()
        base = (s * NC + c) * EPW

        def didx(j, b):
            return pltpu.make_async_copy(
                edges_hbm.at[pl.ds(E + base + j * KD, KD)], cidx_v.at[b],
                isem.at[b])

        def dscat(b):
            return pltpu.make_async_copy(
                ones_v, deg_sh.at[cidx_v.at[b]], ssem.at[b])

        for b in range(3):
            didx(b, b).start()

        def body(g, carry):
            for b in range(NBUF):
                j = g * NBUF + b
                ba = (b + 3) % NBUF

                if b < 2:
                    @pl.when(g >= 1)
                    def _():
                        dscat(ba).wait()

                    didx(j + 3, ba).start()
                else:
                    dscat(ba).wait()

                    @pl.when(g < NGD - 1)
                    def _():
                        didx(j + 3, ba).start()

                didx(j, b).wait()
                dscat(b).start(add=True)
            return carry

        lax.fori_loop(0, NGD, body, 0)
        for t in range(NCHD - 2, NCHD):
            dscat(t % NBUF).wait()
        plsc.subcore_barrier()

        @pl.when(s < NZT)
        def _():
            pltpu.sync_copy(deg_sh.at[pl.ds(s * ZC, ZC)], buf_v)
            pltpu.sync_copy(buf_v, out_hbm.at[pl.ds(c * N + s * ZC, ZC)])

    @functools.partial(
        pl.kernel,
        out_type=jax.ShapeDtypeStruct((2, N, D), jnp.float32),
        mesh=mesh,
        scratch_types=[
            pltpu.VMEM((NIB, K), jnp.int32),
            pltpu.VMEM((NIB, K), jnp.int32),
            pltpu.VMEM((NBUF, K, D), jnp.float32),
            pltpu.VMEM_SHARED((N, D), jnp.float32),
            pltpu.SemaphoreType.DMA((NIB,)),
            pltpu.SemaphoreType.DMA((NBUF,)),
            pltpu.SemaphoreType.DMA((NBUF,)),
        ],
    )
    def prop_kernel(h_hbm, edges_hbm, zeros_hbm, out_hbm,
                    ridx_v, cidx_v, rows_v, acc_sh, isem, gsem, ssem):
        c = lax.axis_index("c")
        s = lax.axis_index("s")
        wid = s * NC + c
        base = wid * EPW
        pltpu.sync_copy(zeros_hbm, rows_v.at[0])

        def zbody(k, carry):
            ch = s + k * NS

            @pl.when(ch < NCHZ)
            def _():
                pltpu.sync_copy(rows_v.at[0], acc_sh.at[pl.ds(ch * ZR, ZR)])

            return carry

        lax.fori_loop(0, JZ, zbody, 0)
        plsc.subcore_barrier()

        def idx(j, bi):
            e0 = base + j * K
            return (pltpu.make_async_copy(edges_hbm.at[pl.ds(e0, K)],
                                          ridx_v.at[bi], isem.at[bi]),
                    pltpu.make_async_copy(edges_hbm.at[pl.ds(E + e0, K)],
                                          cidx_v.at[bi], isem.at[bi]))

        def gath(bi, b):
            return pltpu.make_async_copy(
                h_hbm.at[ridx_v.at[bi]], rows_v.at[b], gsem.at[b])

        def scat(bi, b):
            return pltpu.make_async_copy(
                rows_v.at[b], acc_sh.at[cidx_v.at[bi]], ssem.at[b])

        def idx_start(j, bi):
            d1, d2 = idx(j, bi)
            d1.start()
            d2.start()

        def idx_wait(j, bi):
            d1, d2 = idx(j, bi)
            d1.wait()
            d2.wait()

        for t in range(5):
            idx_start(t, t)
        idx_start(5, 5)
        for t in range(4):
            idx_wait(t, t)
            gath(t, t % NBUF).start()

        def body(g, carry):
            for b in range(NIB):
                j = g * NIB + b
                if b < 4:
                    idx_start(j + 6, (b + 6) % NIB)
                else:
                    @pl.when(g < NG2 - 1)
                    def _():
                        idx_start(j + 6, (b + 6) % NIB)

                b4 = (b + 4) % NBUF
                bi_d = (b + 9) % NIB
                bi_g = (b + 4) % NIB

                def gath_start():
                    idx_wait(j + 4, bi_g)
                    gath(bi_g, b4).start()

                if b < 1:
                    @pl.when(g >= 1)
                    def _():
                        scat(bi_d, b4).wait()

                    gath_start()
                elif b < 6:
                    scat(bi_d, b4).wait()
                    gath_start()
                else:
                    scat(bi_d, b4).wait()

                    @pl.when(g < NG2 - 1)
                    def _():
                        gath_start()

                gath(b % NIB, b % NBUF).wait()
                scat(b % NIB, b % NBUF).start(add=True)
            return carry

        lax.fori_loop(0, NG2, body, 0)
        for t in range(NCH - 1, NCH):
            scat(t % NIB, t % NBUF).wait()
        plsc.subcore_barrier()

        def obody(k, carry):
            ch = s + k * NS

            @pl.when(ch < NCHZ)
            def _():
                pltpu.sync_copy(acc_sh.at[pl.ds(ch * ZR, ZR)], rows_v.at[0])
                pltpu.sync_copy(rows_v.at[0], out_hbm.at[c, pl.ds(ch * ZR, ZR)])

            return carry

        lax.fori_loop(0, JZ, obody, 0)

    RB = 2000
    assert N % RB == 0
    grid = (N // RB,)

    def mm_scale_body(x_ref, w_ref, g_ref, o_ref):
        dinv = _dinv_from(g_ref)
        o_ref[...] = _dot(x_ref[...], w_ref[...]) * dinv[:, None]

    mm_scale = pl.pallas_call(
        mm_scale_body,
        grid=grid,
        in_specs=[
            pl.BlockSpec((RB, D), lambda i: (i, 0)),
            pl.BlockSpec((D, D), lambda i: (0, 0)),
            pl.BlockSpec((RB, 2), lambda i: (i, 0)),
        ],
        out_specs=pl.BlockSpec((RB, D), lambda i: (i, 0)),
        out_shape=jax.ShapeDtypeStruct((N, D), jnp.float32),
    )

    def mid_body(pp_ref, g_ref, b_ref, w_ref, o_ref):
        dinv = _dinv_from(g_ref)
        h1 = (pp_ref[0] + pp_ref[1]) * dinv[:, None] + b_ref[...]
        o_ref[...] = _dot(h1, w_ref[...]) * dinv[:, None]

    mid = pl.pallas_call(
        mid_body,
        grid=grid,
        in_specs=[
            pl.BlockSpec((2, RB, D), lambda i: (0, i, 0)),
            pl.BlockSpec((RB, 2), lambda i: (i, 0)),
            pl.BlockSpec((1, D), lambda i: (0, 0)),
            pl.BlockSpec((D, D), lambda i: (0, 0)),
        ],
        out_specs=pl.BlockSpec((RB, D), lambda i: (i, 0)),
        out_shape=jax.ShapeDtypeStruct((N, D), jnp.float32),
    )

    def fin_body(pp_ref, g_ref, b_ref, o_ref):
        dinv = _dinv_from(g_ref)
        o_ref[...] = (pp_ref[0] + pp_ref[1]) * dinv[:, None] + b_ref[...]

    fin = pl.pallas_call(
        fin_body,
        grid=grid,
        in_specs=[
            pl.BlockSpec((2, RB, D), lambda i: (0, i, 0)),
            pl.BlockSpec((RB, 2), lambda i: (i, 0)),
            pl.BlockSpec((1, D), lambda i: (0, 0)),
        ],
        out_specs=pl.BlockSpec((RB, D), lambda i: (i, 0)),
        out_shape=jax.ShapeDtypeStruct((N, D), jnp.float32),
    )

    return deg_kernel, prop_kernel, mm_scale, mid, fin


def kernel(x, edge_index, W1, b1, W2, b2):
    N, D = x.shape
    E = edge_index.shape[1]
    deg_kernel, prop_kernel, mm_scale, mid, fin = _build(N, D, E)

    EPW = E // NW
    K = 40
    NCH = EPW // K
    edges = edge_index.reshape(2 * E)
    zeros_nd = jnp.zeros((K, D), jnp.float32)
    zeros_n = jnp.zeros((1000,), jnp.float32)
    ones_k = jnp.ones((80,), jnp.float32)

    degp = deg_kernel(edges, zeros_n, ones_k)
    degt = degp.reshape(2, N).T

    h1 = mm_scale(x, W1, degt)
    pp1 = prop_kernel(h1, edges, zeros_nd)
    h2 = mid(pp1, degt, b1.reshape(1, D), W2)
    pp2 = prop_kernel(h2, edges, zeros_nd)
    return fin(pp2, degt, b2.reshape(1, D))


# speedup vs baseline: 1.1316x; 1.0014x over previous
import functools

import jax
import jax.numpy as jnp
from jax import lax
from jax.experimental import pallas as pl
from jax.experimental.pallas import tpu as pltpu
from jax.experimental.pallas import tpu_sc as plsc

NC = 2
NS = 16
NW = NC * NS


def _dinv_from(g_ref):
    deg = g_ref[:, 0] + g_ref[:, 1]
    return jnp.where(deg > 0, lax.rsqrt(deg), 0.0)


def _dot(a, b):
    return jnp.dot(a, b, preferred_element_type=jnp.float32)


@functools.lru_cache(maxsize=None)
def _build(N, D, E):
    EPW = E // NW
    K = 40
    assert EPW % K == 0
    NCH = EPW // K
    NBUF = 5
    NIB = 10
    assert NCH % NIB == 0
    NG2 = NCH // NIB
    NG = NCH // NBUF
    KD = 80
    assert EPW % KD == 0
    NCHD = EPW // KD
    assert NCHD % NBUF == 0
    NGD = NCHD // NBUF
    ZC = 1000
    NZT = N // ZC
    ZR = K
    NCHZ = N // ZR
    JZ = -(-NCHZ // NS)
    assert N % ZR == 0 and ZR % 8 == 0

    mesh = plsc.VectorSubcoreMesh(core_axis_name="c", subcore_axis_name="s")

    @functools.partial(
        pl.kernel,
        out_type=jax.ShapeDtypeStruct((2 * N,), jnp.float32),
        mesh=mesh,
        scratch_types=[
            pltpu.VMEM((NBUF, KD), jnp.int32),
            pltpu.VMEM((KD,), jnp.float32),
            pltpu.VMEM((ZC,), jnp.float32),
            pltpu.VMEM_SHARED((N,), jnp.float32),
            pltpu.SemaphoreType.DMA((NBUF,)),
            pltpu.SemaphoreType.DMA((NBUF,)),
        ],
    )
    def deg_kernel(edges_hbm, zeros_hbm, ones_hbm, out_hbm, cidx_v, ones_v,
                   buf_v, deg_sh, isem, ssem):
        c = lax.axis_index("c")
        s = lax.axis_index("s")
        pltpu.sync_copy(ones_hbm, ones_v)

        @pl.when(s < NZT)
        def _():
            pltpu.sync_copy(zeros_hbm, buf_v)
            pltpu.sync_copy(buf_v, deg_sh.at[pl.ds(s * ZC, ZC)])

        plsc.subcore_barrier()
        base = (s * NC + c) * EPW

        def didx(j, b):
            return pltpu.make_async_copy(
                edges_hbm.at[pl.ds(E + base + j * KD, KD)], cidx_v.at[b],
                isem.at[b])

        def dscat(b):
            return pltpu.make_async_copy(
                ones_v, deg_sh.at[cidx_v.at[b]], ssem.at[b])

        for b in range(4):
            didx(b, b).start()

        def body(g, carry):
            for b in range(NBUF):
                j = g * NBUF + b
                ba = (b + 4) % NBUF

                if b < 1:
                    @pl.when(g >= 1)
                    def _():
                        dscat(ba).wait()

                    didx(j + 4, ba).start()
                else:
                    dscat(ba).wait()

                    @pl.when(g < NGD - 1)
                    def _():
                        didx(j + 4, ba).start()

                didx(j, b).wait()
                dscat(b).start(add=True)
            return carry

        lax.fori_loop(0, NGD, body, 0)
        dscat((NCHD - 1) % NBUF).wait()
        plsc.subcore_barrier()

        @pl.when(s < NZT)
        def _():
            pltpu.sync_copy(deg_sh.at[pl.ds(s * ZC, ZC)], buf_v)
            pltpu.sync_copy(buf_v, out_hbm.at[pl.ds(c * N + s * ZC, ZC)])

    @functools.partial(
        pl.kernel,
        out_type=jax.ShapeDtypeStruct((2, N, D), jnp.float32),
        mesh=mesh,
        scratch_types=[
            pltpu.VMEM((NIB, K), jnp.int32),
            pltpu.VMEM((NIB, K), jnp.int32),
            pltpu.VMEM((NBUF, K, D), jnp.float32),
            pltpu.VMEM_SHARED((N, D), jnp.float32),
            pltpu.SemaphoreType.DMA((NIB,)),
            pltpu.SemaphoreType.DMA((NBUF,)),
            pltpu.SemaphoreType.DMA((NBUF,)),
        ],
    )
    def prop_kernel(h_hbm, edges_hbm, zeros_hbm, out_hbm,
                    ridx_v, cidx_v, rows_v, acc_sh, isem, gsem, ssem):
        c = lax.axis_index("c")
        s = lax.axis_index("s")
        wid = s * NC + c
        base = wid * EPW
        pltpu.sync_copy(zeros_hbm, rows_v.at[0])

        def zbody(k, carry):
            ch = s + k * NS

            @pl.when(ch < NCHZ)
            def _():
                pltpu.sync_copy(rows_v.at[0], acc_sh.at[pl.ds(ch * ZR, ZR)])

            return carry

        lax.fori_loop(0, JZ, zbody, 0)
        plsc.subcore_barrier()

        def idx(j, bi):
            e0 = base + j * K
            return (pltpu.make_async_copy(edges_hbm.at[pl.ds(e0, K)],
                                          ridx_v.at[bi], isem.at[bi]),
                    pltpu.make_async_copy(edges_hbm.at[pl.ds(E + e0, K)],
                                          cidx_v.at[bi], isem.at[bi]))

        def gath(bi, b):
            return pltpu.make_async_copy(
                h_hbm.at[ridx_v.at[bi]], rows_v.at[b], gsem.at[b])

        def scat(bi, b):
            return pltpu.make_async_copy(
                rows_v.at[b], acc_sh.at[cidx_v.at[bi]], ssem.at[b])

        def idx_start(j, bi):
            d1, d2 = idx(j, bi)
            d1.start()
            d2.start()

        def idx_wait(j, bi):
            d1, d2 = idx(j, bi)
            d1.wait()
            d2.wait()

        for t in range(5):
            idx_start(t, t)
        idx_start(5, 5)
        for t in range(4):
            idx_wait(t, t)
            gath(t, t % NBUF).start()

        def body(g, carry):
            for b in range(NIB):
                j = g * NIB + b
                if b < 4:
                    idx_start(j + 6, (b + 6) % NIB)
                else:
                    @pl.when(g < NG2 - 1)
                    def _():
                        idx_start(j + 6, (b + 6) % NIB)

                b4 = (b + 4) % NBUF
                bi_d = (b + 9) % NIB
                bi_g = (b + 4) % NIB

                def gath_start():
                    idx_wait(j + 4, bi_g)
                    gath(bi_g, b4).start()

                if b < 1:
                    @pl.when(g >= 1)
                    def _():
                        scat(bi_d, b4).wait()

                    gath_start()
                elif b < 6:
                    scat(bi_d, b4).wait()
                    gath_start()
                else:
                    scat(bi_d, b4).wait()

                    @pl.when(g < NG2 - 1)
                    def _():
                        gath_start()

                gath(b % NIB, b % NBUF).wait()
                scat(b % NIB, b % NBUF).start(add=True)
            return carry

        lax.fori_loop(0, NG2, body, 0)
        for t in range(NCH - 1, NCH):
            scat(t % NIB, t % NBUF).wait()
        plsc.subcore_barrier()

        def obody(k, carry):
            ch = s + k * NS

            @pl.when(ch < NCHZ)
            def _():
                pltpu.sync_copy(acc_sh.at[pl.ds(ch * ZR, ZR)], rows_v.at[0])
                pltpu.sync_copy(rows_v.at[0], out_hbm.at[c, pl.ds(ch * ZR, ZR)])

            return carry

        lax.fori_loop(0, JZ, obody, 0)

    RB = 2000
    assert N % RB == 0
    grid = (N // RB,)

    def mm_scale_body(x_ref, w_ref, g_ref, o_ref):
        dinv = _dinv_from(g_ref)
        o_ref[...] = _dot(x_ref[...], w_ref[...]) * dinv[:, None]

    mm_scale = pl.pallas_call(
        mm_scale_body,
        grid=grid,
        in_specs=[
            pl.BlockSpec((RB, D), lambda i: (i, 0)),
            pl.BlockSpec((D, D), lambda i: (0, 0)),
            pl.BlockSpec((RB, 2), lambda i: (i, 0)),
        ],
        out_specs=pl.BlockSpec((RB, D), lambda i: (i, 0)),
        out_shape=jax.ShapeDtypeStruct((N, D), jnp.float32),
    )

    def mid_body(pp_ref, g_ref, b_ref, w_ref, o_ref):
        dinv = _dinv_from(g_ref)
        h1 = (pp_ref[0] + pp_ref[1]) * dinv[:, None] + b_ref[...]
        o_ref[...] = _dot(h1, w_ref[...]) * dinv[:, None]

    mid = pl.pallas_call(
        mid_body,
        grid=grid,
        in_specs=[
            pl.BlockSpec((2, RB, D), lambda i: (0, i, 0)),
            pl.BlockSpec((RB, 2), lambda i: (i, 0)),
            pl.BlockSpec((1, D), lambda i: (0, 0)),
            pl.BlockSpec((D, D), lambda i: (0, 0)),
        ],
        out_specs=pl.BlockSpec((RB, D), lambda i: (i, 0)),
        out_shape=jax.ShapeDtypeStruct((N, D), jnp.float32),
    )

    def fin_body(pp_ref, g_ref, b_ref, o_ref):
        dinv = _dinv_from(g_ref)
        o_ref[...] = (pp_ref[0] + pp_ref[1]) * dinv[:, None] + b_ref[...]

    fin = pl.pallas_call(
        fin_body,
        grid=grid,
        in_specs=[
            pl.BlockSpec((2, RB, D), lambda i: (0, i, 0)),
            pl.BlockSpec((RB, 2), lambda i: (i, 0)),
            pl.BlockSpec((1, D), lambda i: (0, 0)),
        ],
        out_specs=pl.BlockSpec((RB, D), lambda i: (i, 0)),
        out_shape=jax.ShapeDtypeStruct((N, D), jnp.float32),
    )

    return deg_kernel, prop_kernel, mm_scale, mid, fin


def kernel(x, edge_index, W1, b1, W2, b2):
    N, D = x.shape
    E = edge_index.shape[1]
    deg_kernel, prop_kernel, mm_scale, mid, fin = _build(N, D, E)

    EPW = E // NW
    K = 40
    NCH = EPW // K
    edges = edge_index.reshape(2 * E)
    zeros_nd = jnp.zeros((K, D), jnp.float32)
    zeros_n = jnp.zeros((1000,), jnp.float32)
    ones_k = jnp.ones((80,), jnp.float32)

    degp = deg_kernel(edges, zeros_n, ones_k)
    degt = degp.reshape(2, N).T

    h1 = mm_scale(x, W1, degt)
    pp1 = prop_kernel(h1, edges, zeros_nd)
    h2 = mid(pp1, degt, b1.reshape(1, D), W2)
    pp2 = prop_kernel(h2, edges, zeros_nd)
    return fin(pp2, degt, b2.reshape(1, D))
